# baseline (speedup 1.0000x reference)
import jax
import jax.numpy as jnp
from jax.experimental import pallas as pl

_B, _P, _C, _O = 32, 20000, 21, 16
_R, _L = 8, 2500
_G = 4
_JT = 0.5
_NEG = 3
_V0, _V1 = 0.1, 0.2


def _one_batch(loc, conf, tgt, priors, lin):
    f32 = jnp.float32
    pcx, pcy, pw, ph, px0, py0, px1, py1, area_b = priors

    ovs = []
    for j in range(_O):
        tx0 = tgt[j, 0]
        ty0 = tgt[j, 1]
        tx1 = tgt[j, 2]
        ty1 = tgt[j, 3]
        iw = jnp.maximum(jnp.minimum(tx1, px1) - jnp.maximum(tx0, px0), 0.0)
        ih = jnp.maximum(jnp.minimum(ty1, py1) - jnp.maximum(ty0, py0), 0.0)
        inter = iw * ih
        area_a = (tx1 - tx0) * (ty1 - ty0)
        ovs.append(inter / (area_a + area_b - inter))
    ov = jnp.stack(ovs)

    jio = jax.lax.broadcasted_iota(jnp.int32, (_O, _R, _L), 0)
    lin3 = (jax.lax.broadcasted_iota(jnp.int32, (_O, _R, _L), 1) * _L
            + jax.lax.broadcasted_iota(jnp.int32, (_O, _R, _L), 2))

    bto = jnp.max(ov, axis=0)
    bti = jnp.min(jnp.where(ov == bto[None], jio, _O), axis=0)

    mxj = jnp.max(ov, axis=(1, 2))
    bpi = jnp.min(jnp.where(ov == mxj[:, None, None], lin3, _P),
                  axis=(1, 2))

    hit = lin3 == bpi[:, None, None]
    any_hit = jnp.max(hit.astype(jnp.int32), axis=0) > 0
    jwin = jnp.max(jnp.where(hit, jio, -1), axis=0)
    bto = jnp.where(any_hit, 2.0, bto)
    bti = jnp.where(any_hit, jwin, bti)

    sel = bti[None] == jio
    tcol = [tgt[:, k].reshape(_O, 1, 1) for k in range(5)]
    mx0 = jnp.max(jnp.where(sel, tcol[0], -1e30), axis=0)
    my0 = jnp.max(jnp.where(sel, tcol[1], -1e30), axis=0)
    mx1 = jnp.max(jnp.where(sel, tcol[2], -1e30), axis=0)
    my1 = jnp.max(jnp.where(sel, tcol[3], -1e30), axis=0)
    lbl = jnp.max(jnp.where(sel, tcol[4], -1e30), axis=0)

    pos = bto >= _JT
    posf = pos.astype(f32)
    conf_lbl = jnp.where(pos, lbl.astype(jnp.int32) + 1, 0)

    g = (
        ((mx0 + mx1) * 0.5 - pcx) / (_V0 * pw),
        ((my0 + my1) * 0.5 - pcy) / (_V0 * ph),
        jnp.log((mx1 - mx0) / pw) / _V1,
        jnp.log((my1 - my0) / ph) / _V1,
    )
    ll = jnp.float32(0.0)
    for k in range(4):
        d = loc[k] - g[k]
        ad = jnp.abs(d)
        s = jnp.where(ad < 1.0, 0.5 * d * d, ad - 0.5)
        ll = ll + jnp.sum(s * posf)

    mx = jnp.max(conf, axis=0)
    se = jnp.sum(jnp.exp(conf - mx[None]), axis=0)
    cio = jax.lax.broadcasted_iota(jnp.int32, (_C, _R, _L), 0)
    xl = jnp.sum(jnp.where(cio == conf_lbl[None], conf, 0.0), axis=0)
    ce = mx + jnp.log(se) - xl

    npos = jnp.sum(pos.astype(jnp.int32))
    ce_pos = jnp.sum(ce * posf)
    mce = jnp.where(pos, 0.0, ce)
    kk = jnp.minimum(npos * _NEG, _P)
    xb = jax.lax.bitcast_convert_type(mce, jnp.int32)
    return ll, ce_pos, npos, kk, xb, mce


def _mbl_kernel(loc_ref, conf_ref, dbox_ref, tgt_ref, out_ref):
    f32 = jnp.float32
    b = pl.program_id(0)

    pr = dbox_ref[...]
    pcx, pcy, pw, ph = pr[0], pr[1], pr[2], pr[3]
    px0 = pcx - pw * 0.5
    py0 = pcy - ph * 0.5
    px1 = pcx + pw * 0.5
    py1 = pcy + ph * 0.5
    area_b = (px1 - px0) * (py1 - py0)
    priors = (pcx, pcy, pw, ph, px0, py0, px1, py1, area_b)

    lin = (jax.lax.broadcasted_iota(jnp.int32, (_R, _L), 0) * _L
           + jax.lax.broadcasted_iota(jnp.int32, (_R, _L), 1))

    st = [_one_batch(loc_ref[gi], conf_ref[gi], tgt_ref[gi], priors, lin)
          for gi in range(_G)]

    def body(_, lohi):
        lo, hi = lohi
        nlo, nhi = [], []
        for gi in range(_G):
            mid = lo[gi] + (hi[gi] - lo[gi]) // 2
            cnt = jnp.sum((st[gi][4] >= mid).astype(jnp.int32))
            ok = cnt >= st[gi][3]
            nlo.append(jnp.where(ok, mid, lo[gi]))
            nhi.append(jnp.where(ok, hi[gi], mid))
        return tuple(nlo), tuple(nhi)

    init = (tuple(jnp.int32(0) for _ in range(_G)),
            tuple(jnp.int32(0x7F800000) for _ in range(_G)))
    lo, _hi = jax.lax.fori_loop(0, 31, body, init)

    ll_t = jnp.float32(0.0)
    lc_t = jnp.float32(0.0)
    np_t = jnp.int32(0)
    for gi in range(_G):
        ll, ce_pos, npos, kk, _xb, mce = st[gi]
        v = jax.lax.bitcast_convert_type(lo[gi], f32)
        gt = mce > v
        cnt_gt = jnp.sum(gt.astype(jnp.int32))
        sum_gt = jnp.sum(jnp.where(gt, mce, 0.0))
        topk = sum_gt + (kk - cnt_gt).astype(f32) * v
        ll_t = ll_t + ll
        lc_t = lc_t + ce_pos + topk
        np_t = np_t + npos

    @pl.when(b == 0)
    def _init():
        out_ref[...] = jnp.zeros_like(out_ref)

    io = jax.lax.broadcasted_iota(jnp.int32, (1, 8), 1)
    vals = (jnp.where(io == 0, ll_t, 0.0)
            + jnp.where(io == 1, lc_t, 0.0)
            + jnp.where(io == 2, np_t.astype(f32), 0.0))
    out_ref[0] = out_ref[0] + vals


def kernel(loc_data, conf_data, dbox_list, targets):
    loc_t4 = loc_data.transpose(0, 2, 1).reshape(_B, 4, _R, _L)
    conf_r = conf_data.transpose(0, 2, 1).reshape(_B, _C, _R, _L)
    dbox_s = dbox_list.T.reshape(4, _R, _L)

    out = pl.pallas_call(
        _mbl_kernel,
        grid=(_B // _G,),
        in_specs=[
            pl.BlockSpec((_G, 4, _R, _L), lambda b: (b, 0, 0, 0)),
            pl.BlockSpec((_G, _C, _R, _L), lambda b: (b, 0, 0, 0)),
            pl.BlockSpec((4, _R, _L), lambda b: (0, 0, 0)),
            pl.BlockSpec((_G, _O, 5), lambda b: (b, 0, 0)),
        ],
        out_specs=pl.BlockSpec((1, 1, 8), lambda b: (0, 0, 0)),
        out_shape=jax.ShapeDtypeStruct((1, 1, 8), jnp.float32),
    )(loc_t4, conf_r, dbox_s, targets)

    ll = out[0, 0, 0]
    lc = out[0, 0, 1]
    n = out[0, 0, 2]
    z = jnp.sum(conf_data) * 0.0
    return ll / n + z, lc / n


# speedup vs baseline: 1.1322x; 1.1322x over previous
import jax
import jax.numpy as jnp
from jax.experimental import pallas as pl

_B, _P, _C, _O = 32, 20000, 21, 16
_R, _L = 8, 2500
_G = 8
_JT = 0.5
_NEG = 3
_V0, _V1 = 0.1, 0.2


def _one_batch(loc, conf, tgt, priors, lin):
    f32 = jnp.float32
    pcx, pcy, pw, ph, px0, py0, px1, py1, area_b = priors

    ovs = []
    for j in range(_O):
        tx0 = tgt[j, 0]
        ty0 = tgt[j, 1]
        tx1 = tgt[j, 2]
        ty1 = tgt[j, 3]
        iw = jnp.maximum(jnp.minimum(tx1, px1) - jnp.maximum(tx0, px0), 0.0)
        ih = jnp.maximum(jnp.minimum(ty1, py1) - jnp.maximum(ty0, py0), 0.0)
        inter = iw * ih
        area_a = (tx1 - tx0) * (ty1 - ty0)
        ovs.append(inter / (area_a + area_b - inter))
    ov = jnp.stack(ovs)

    jio = jax.lax.broadcasted_iota(jnp.int32, (_O, _R, _L), 0)
    lin3 = (jax.lax.broadcasted_iota(jnp.int32, (_O, _R, _L), 1) * _L
            + jax.lax.broadcasted_iota(jnp.int32, (_O, _R, _L), 2))

    bto = jnp.max(ov, axis=0)
    bti = jnp.min(jnp.where(ov == bto[None], jio, _O), axis=0)

    mxj = jnp.max(ov, axis=(1, 2))
    bpi = jnp.min(jnp.where(ov == mxj[:, None, None], lin3, _P),
                  axis=(1, 2))

    hit = lin3 == bpi[:, None, None]
    any_hit = jnp.max(hit.astype(jnp.int32), axis=0) > 0
    jwin = jnp.max(jnp.where(hit, jio, -1), axis=0)
    bto = jnp.where(any_hit, 2.0, bto)
    bti = jnp.where(any_hit, jwin, bti)

    sel = bti[None] == jio
    tcol = [tgt[:, k].reshape(_O, 1, 1) for k in range(5)]
    mx0 = jnp.max(jnp.where(sel, tcol[0], -1e30), axis=0)
    my0 = jnp.max(jnp.where(sel, tcol[1], -1e30), axis=0)
    mx1 = jnp.max(jnp.where(sel, tcol[2], -1e30), axis=0)
    my1 = jnp.max(jnp.where(sel, tcol[3], -1e30), axis=0)
    lbl = jnp.max(jnp.where(sel, tcol[4], -1e30), axis=0)

    pos = bto >= _JT
    posf = pos.astype(f32)
    conf_lbl = jnp.where(pos, lbl.astype(jnp.int32) + 1, 0)

    g = (
        ((mx0 + mx1) * 0.5 - pcx) / (_V0 * pw),
        ((my0 + my1) * 0.5 - pcy) / (_V0 * ph),
        jnp.log((mx1 - mx0) / pw) / _V1,
        jnp.log((my1 - my0) / ph) / _V1,
    )
    ll = jnp.float32(0.0)
    for k in range(4):
        d = loc[k] - g[k]
        ad = jnp.abs(d)
        s = jnp.where(ad < 1.0, 0.5 * d * d, ad - 0.5)
        ll = ll + jnp.sum(s * posf)

    mx = jnp.max(conf, axis=0)
    se = jnp.sum(jnp.exp(conf - mx[None]), axis=0)
    cio = jax.lax.broadcasted_iota(jnp.int32, (_C, _R, _L), 0)
    xl = jnp.sum(jnp.where(cio == conf_lbl[None], conf, 0.0), axis=0)
    ce = mx + jnp.log(se) - xl

    npos = jnp.sum(pos.astype(jnp.int32))
    ce_pos = jnp.sum(ce * posf)
    mce = jnp.where(pos, 0.0, ce)
    kk = jnp.minimum(npos * _NEG, _P)
    xb = jax.lax.bitcast_convert_type(mce, jnp.int32)
    return ll, ce_pos, npos, kk, xb, mce


def _mbl_kernel(loc_ref, conf_ref, dbox_ref, tgt_ref, out_ref):
    f32 = jnp.float32
    b = pl.program_id(0)

    pr = dbox_ref[...]
    pcx, pcy, pw, ph = pr[0], pr[1], pr[2], pr[3]
    px0 = pcx - pw * 0.5
    py0 = pcy - ph * 0.5
    px1 = pcx + pw * 0.5
    py1 = pcy + ph * 0.5
    area_b = (px1 - px0) * (py1 - py0)
    priors = (pcx, pcy, pw, ph, px0, py0, px1, py1, area_b)

    lin = (jax.lax.broadcasted_iota(jnp.int32, (_R, _L), 0) * _L
           + jax.lax.broadcasted_iota(jnp.int32, (_R, _L), 1))

    st = [_one_batch(loc_ref[gi], conf_ref[gi], tgt_ref[gi], priors, lin)
          for gi in range(_G)]

    def body(_, lohi):
        lo, hi = lohi
        nlo, nhi = [], []
        for gi in range(_G):
            mid = lo[gi] + (hi[gi] - lo[gi]) // 2
            cnt = jnp.sum((st[gi][4] >= mid).astype(jnp.int32))
            ok = cnt >= st[gi][3]
            nlo.append(jnp.where(ok, mid, lo[gi]))
            nhi.append(jnp.where(ok, hi[gi], mid))
        return tuple(nlo), tuple(nhi)

    init = (tuple(jnp.int32(0) for _ in range(_G)),
            tuple(jnp.int32(0x7F800000) for _ in range(_G)))
    lo, _hi = jax.lax.fori_loop(0, 31, body, init)

    ll_t = jnp.float32(0.0)
    lc_t = jnp.float32(0.0)
    np_t = jnp.int32(0)
    for gi in range(_G):
        ll, ce_pos, npos, kk, _xb, mce = st[gi]
        v = jax.lax.bitcast_convert_type(lo[gi], f32)
        gt = mce > v
        cnt_gt = jnp.sum(gt.astype(jnp.int32))
        sum_gt = jnp.sum(jnp.where(gt, mce, 0.0))
        topk = sum_gt + (kk - cnt_gt).astype(f32) * v
        ll_t = ll_t + ll
        lc_t = lc_t + ce_pos + topk
        np_t = np_t + npos

    @pl.when(b == 0)
    def _init():
        out_ref[...] = jnp.zeros_like(out_ref)

    io = jax.lax.broadcasted_iota(jnp.int32, (1, 8), 1)
    vals = (jnp.where(io == 0, ll_t, 0.0)
            + jnp.where(io == 1, lc_t, 0.0)
            + jnp.where(io == 2, np_t.astype(f32), 0.0))
    out_ref[0] = out_ref[0] + vals


def kernel(loc_data, conf_data, dbox_list, targets):
    loc_t4 = loc_data.transpose(0, 2, 1).reshape(_B, 4, _R, _L)
    conf_r = conf_data.transpose(0, 2, 1).reshape(_B, _C, _R, _L)
    dbox_s = dbox_list.T.reshape(4, _R, _L)

    out = pl.pallas_call(
        _mbl_kernel,
        grid=(_B // _G,),
        in_specs=[
            pl.BlockSpec((_G, 4, _R, _L), lambda b: (b, 0, 0, 0)),
            pl.BlockSpec((_G, _C, _R, _L), lambda b: (b, 0, 0, 0)),
            pl.BlockSpec((4, _R, _L), lambda b: (0, 0, 0)),
            pl.BlockSpec((_G, _O, 5), lambda b: (b, 0, 0)),
        ],
        out_specs=pl.BlockSpec((1, 1, 8), lambda b: (0, 0, 0)),
        out_shape=jax.ShapeDtypeStruct((1, 1, 8), jnp.float32),
    )(loc_t4, conf_r, dbox_s, targets)

    ll = out[0, 0, 0]
    lc = out[0, 0, 1]
    n = out[0, 0, 2]
    return ll / n, lc / n


# speedup vs baseline: 1.2980x; 1.1464x over previous
import jax
import jax.numpy as jnp
from jax.experimental import pallas as pl

_B, _P, _C, _O = 32, 20000, 21, 16
_R, _L = 8, 2500
_G = 8
_JT = 0.5
_NEG = 3
_V0, _V1 = 0.1, 0.2


def _one_batch(loc, conf, tgt, priors, lin):
    f32 = jnp.float32
    pcx, pcy, pw, ph, px0, py0, px1, py1, area_b = priors

    ovs = []
    for j in range(_O):
        tx0 = tgt[j, 0]
        ty0 = tgt[j, 1]
        tx1 = tgt[j, 2]
        ty1 = tgt[j, 3]
        iw = jnp.maximum(jnp.minimum(tx1, px1) - jnp.maximum(tx0, px0), 0.0)
        ih = jnp.maximum(jnp.minimum(ty1, py1) - jnp.maximum(ty0, py0), 0.0)
        inter = iw * ih
        area_a = (tx1 - tx0) * (ty1 - ty0)
        ovs.append(inter / (area_a + area_b - inter))
    ov = jnp.stack(ovs)

    jio = jax.lax.broadcasted_iota(jnp.int32, (_O, _R, _L), 0)
    lin3 = (jax.lax.broadcasted_iota(jnp.int32, (_O, _R, _L), 1) * _L
            + jax.lax.broadcasted_iota(jnp.int32, (_O, _R, _L), 2))

    bto = jnp.max(ov, axis=0)
    bti = jnp.min(jnp.where(ov == bto[None], jio, _O), axis=0)

    mxj = jnp.max(ov, axis=(1, 2))
    bpi = jnp.min(jnp.where(ov == mxj[:, None, None], lin3, _P),
                  axis=(1, 2))

    hit = lin3 == bpi[:, None, None]
    any_hit = jnp.max(hit.astype(jnp.int32), axis=0) > 0
    jwin = jnp.max(jnp.where(hit, jio, -1), axis=0)
    bto = jnp.where(any_hit, 2.0, bto)
    bti = jnp.where(any_hit, jwin, bti)

    sel = bti[None] == jio
    tcol = [tgt[:, k].reshape(_O, 1, 1) for k in range(5)]
    mx0 = jnp.max(jnp.where(sel, tcol[0], -1e30), axis=0)
    my0 = jnp.max(jnp.where(sel, tcol[1], -1e30), axis=0)
    mx1 = jnp.max(jnp.where(sel, tcol[2], -1e30), axis=0)
    my1 = jnp.max(jnp.where(sel, tcol[3], -1e30), axis=0)
    lbl = jnp.max(jnp.where(sel, tcol[4], -1e30), axis=0)

    pos = bto >= _JT
    posf = pos.astype(f32)
    conf_lbl = jnp.where(pos, lbl.astype(jnp.int32) + 1, 0)

    g = (
        ((mx0 + mx1) * 0.5 - pcx) / (_V0 * pw),
        ((my0 + my1) * 0.5 - pcy) / (_V0 * ph),
        jnp.log((mx1 - mx0) / pw) / _V1,
        jnp.log((my1 - my0) / ph) / _V1,
    )
    ll = jnp.float32(0.0)
    for k in range(4):
        d = loc[k] - g[k]
        ad = jnp.abs(d)
        s = jnp.where(ad < 1.0, 0.5 * d * d, ad - 0.5)
        ll = ll + jnp.sum(s * posf)

    mx = jnp.max(conf, axis=0)
    se = jnp.sum(jnp.exp(conf - mx[None]), axis=0)
    cio = jax.lax.broadcasted_iota(jnp.int32, (_C, _R, _L), 0)
    xl = jnp.sum(jnp.where(cio == conf_lbl[None], conf, 0.0), axis=0)
    ce = mx + jnp.log(se) - xl

    npos = jnp.sum(pos.astype(jnp.int32))
    ce_pos = jnp.sum(ce * posf)
    mce = jnp.where(pos, 0.0, ce)
    kk = jnp.minimum(npos * _NEG, _P)
    xb = jax.lax.bitcast_convert_type(mce, jnp.int32)
    return ll, ce_pos, npos, kk, xb, mce


def _mbl_kernel(loc_ref, conf_ref, dbox_ref, tgt_ref, out_ref):
    f32 = jnp.float32
    b = pl.program_id(0)

    pr = dbox_ref[...]
    pcx, pcy, pw, ph = pr[0], pr[1], pr[2], pr[3]
    px0 = pcx - pw * 0.5
    py0 = pcy - ph * 0.5
    px1 = pcx + pw * 0.5
    py1 = pcy + ph * 0.5
    area_b = (px1 - px0) * (py1 - py0)
    priors = (pcx, pcy, pw, ph, px0, py0, px1, py1, area_b)

    lin = (jax.lax.broadcasted_iota(jnp.int32, (_R, _L), 0) * _L
           + jax.lax.broadcasted_iota(jnp.int32, (_R, _L), 1))

    st = [_one_batch(loc_ref[gi], conf_ref[:, gi], tgt_ref[gi], priors, lin)
          for gi in range(_G)]

    def body(_, lohi):
        lo, hi = lohi
        nlo, nhi = [], []
        for gi in range(_G):
            mid = lo[gi] + (hi[gi] - lo[gi]) // 2
            cnt = jnp.sum((st[gi][4] >= mid).astype(jnp.int32))
            ok = cnt >= st[gi][3]
            nlo.append(jnp.where(ok, mid, lo[gi]))
            nhi.append(jnp.where(ok, hi[gi], mid))
        return tuple(nlo), tuple(nhi)

    init = (tuple(jnp.int32(0) for _ in range(_G)),
            tuple(jnp.int32(0x7F800000) for _ in range(_G)))
    lo, _hi = jax.lax.fori_loop(0, 31, body, init)

    ll_t = jnp.float32(0.0)
    lc_t = jnp.float32(0.0)
    np_t = jnp.int32(0)
    for gi in range(_G):
        ll, ce_pos, npos, kk, _xb, mce = st[gi]
        v = jax.lax.bitcast_convert_type(lo[gi], f32)
        gt = mce > v
        cnt_gt = jnp.sum(gt.astype(jnp.int32))
        sum_gt = jnp.sum(jnp.where(gt, mce, 0.0))
        topk = sum_gt + (kk - cnt_gt).astype(f32) * v
        ll_t = ll_t + ll
        lc_t = lc_t + ce_pos + topk
        np_t = np_t + npos

    @pl.when(b == 0)
    def _init():
        out_ref[...] = jnp.zeros_like(out_ref)

    io = jax.lax.broadcasted_iota(jnp.int32, (1, 8), 1)
    vals = (jnp.where(io == 0, ll_t, 0.0)
            + jnp.where(io == 1, lc_t, 0.0)
            + jnp.where(io == 2, np_t.astype(f32), 0.0))
    out_ref[0] = out_ref[0] + vals


def kernel(loc_data, conf_data, dbox_list, targets):
    loc_t4 = loc_data.transpose(0, 2, 1).reshape(_B, 4, _R, _L)
    conf_r = conf_data.transpose(2, 0, 1).reshape(_C, _B, _R, _L)
    dbox_s = dbox_list.T.reshape(4, _R, _L)

    out = pl.pallas_call(
        _mbl_kernel,
        grid=(_B // _G,),
        in_specs=[
            pl.BlockSpec((_G, 4, _R, _L), lambda b: (b, 0, 0, 0)),
            pl.BlockSpec((_C, _G, _R, _L), lambda b: (0, b, 0, 0)),
            pl.BlockSpec((4, _R, _L), lambda b: (0, 0, 0)),
            pl.BlockSpec((_G, _O, 5), lambda b: (b, 0, 0)),
        ],
        out_specs=pl.BlockSpec((1, 1, 8), lambda b: (0, 0, 0)),
        out_shape=jax.ShapeDtypeStruct((1, 1, 8), jnp.float32),
    )(loc_t4, conf_r, dbox_s, targets)

    ll = out[0, 0, 0]
    lc = out[0, 0, 1]
    n = out[0, 0, 2]
    return ll / n, lc / n
